# baseline (speedup 1.0000x reference)
import functools

import jax
import jax.numpy as jnp
from jax import lax
from jax.experimental import pallas as pl
from jax.experimental.pallas import tpu as pltpu
from jax.experimental.pallas import tpu_sc as plsc

EMBED_DIM = 64
GROUPS_PER_STEP = 32
LANES_PER_STEP = GROUPS_PER_STEP * 512


def _pack_pair(a, b):
    ua = lax.bitcast_convert_type(a, jnp.uint32)
    ub = lax.bitcast_convert_type(b, jnp.uint32)
    mask_hi = jnp.uint32(0xFFFF0000)
    return lax.bitcast_convert_type((ua >> 16) | (ub & mask_hi),
                                    jnp.float32)


def _repack_body(t_ref, o_ref):
    for g in range(GROUPS_PER_STEP):
        base = g * 512
        t01 = jnp.concatenate(
            [t_ref[:, base:base + 128], t_ref[:, base + 128:base + 256]],
            axis=0).T
        t23 = jnp.concatenate(
            [t_ref[:, base + 256:base + 384], t_ref[:, base + 384:base + 512]],
            axis=0).T
        lo = _pack_pair(t01[:, :EMBED_DIM], t01[:, EMBED_DIM:])
        hi = _pack_pair(t23[:, :EMBED_DIM], t23[:, EMBED_DIM:])
        o_ref[g * 128:(g + 1) * 128, :] = jnp.concatenate([lo, hi], axis=1)


def _repack(tabT, n_steps):
    return pl.pallas_call(
        _repack_body,
        grid=(n_steps,),
        in_specs=[pl.BlockSpec((EMBED_DIM, LANES_PER_STEP), lambda i: (0, i))],
        out_specs=pl.BlockSpec((GROUPS_PER_STEP * 128, 2 * EMBED_DIM),
                               lambda i: (i, 0)),
        out_shape=jax.ShapeDtypeStruct(
            (n_steps * GROUPS_PER_STEP * 128, 2 * EMBED_DIM), jnp.float32),
    )(tabT)


@functools.cache
def _gather_fn(B, Np):
    info = plsc.get_sparse_core_info()
    NC, NS = info.num_cores, info.num_subcores
    NW = NC * NS
    b_per_w = B // NW
    CH = 128
    k = b_per_w // CH
    mesh = plsc.VectorSubcoreMesh(core_axis_name="c", subcore_axis_name="s")

    @functools.partial(
        pl.kernel,
        mesh=mesh,
        compiler_params=pltpu.CompilerParams(
            use_tc_tiling_on_sc=True, needs_layout_passes=False),
        out_type=jax.ShapeDtypeStruct((B, 2 * EMBED_DIM), jnp.float32),
        scratch_types=[
            pltpu.VMEM((k, CH), jnp.int32),
            pltpu.VMEM((CH, 2 * EMBED_DIM), jnp.float32),
            pltpu.VMEM((CH, 2 * EMBED_DIM), jnp.float32),
            pltpu.SemaphoreType.DMA,
            pltpu.SemaphoreType.DMA,
            pltpu.SemaphoreType.DMA,
            pltpu.SemaphoreType.DMA,
        ],
    )
    def gk(idx_hbm, tab, out, idx_v, buf0, buf1, gsem0, gsem1, wsem0, wsem1):
        wid = lax.axis_index("s") * NC + lax.axis_index("c")
        base = wid * b_per_w
        pltpu.sync_copy(idx_hbm.at[pl.ds(wid * k, k)], idx_v)
        bufs = (buf0, buf1)
        gsems = (gsem0, gsem1)
        wsems = (wsem0, wsem1)
        gathers = [None, None]
        writes = [None, None]
        for i in range(k):
            p = i % 2
            if writes[p] is not None:
                writes[p].wait()
            gathers[p] = pltpu.async_copy(
                tab.at[idx_v.at[i]], bufs[p], gsems[p])
            if i >= 1:
                q = (i - 1) % 2
                gathers[q].wait()
                writes[q] = pltpu.async_copy(
                    bufs[q], out.at[pl.ds(base + (i - 1) * CH, CH)], wsems[q])
        last = (k - 1) % 2
        gathers[last].wait()
        writes[last] = pltpu.async_copy(
            bufs[last], out.at[pl.ds(base + (k - 1) * CH, CH)], wsems[last])
        for p in range(2):
            if writes[p] is not None:
                writes[p].wait()

    return gk


def _unpack_select(g, q):
    gh = jnp.where(q >= 2, g[:, EMBED_DIM:], g[:, :EMBED_DIM])
    v = lax.bitcast_convert_type(gh, jnp.uint32)
    lo = lax.bitcast_convert_type(
        (v & 0xFFFF).astype(jnp.uint16), jnp.bfloat16)
    hi = lax.bitcast_convert_type(
        (v >> 16).astype(jnp.uint16), jnp.bfloat16)
    return jnp.where(q % 2 == 1, hi, lo)


def _mlp_body(gu, gp, qu, qp, w1u, w1p, b1, w2, b2, w3r, b3, o):
    f32 = jnp.float32
    u = _unpack_select(gu[...], qu[...])
    p = _unpack_select(gp[...], qp[...])
    cn = (((1,), (0,)), ((), ()))
    h = (lax.dot_general(u, w1u[...], cn, preferred_element_type=f32)
         + lax.dot_general(p, w1p[...], cn, preferred_element_type=f32)
         + b1[...])
    h = jnp.maximum(h, 0.0)
    h = jnp.maximum(jnp.dot(h, w2[...]) + b2[...], 0.0)
    cn11 = (((1,), (1,)), ((), ()))
    o[...] = jax.nn.sigmoid(lax.dot_general(w3r[...], h, cn11) + b3[...])


def _mlp(gu, gp, qu, qp, W1u, W1p, b1, W2, b2, W3r, b3):
    B = gu.shape[0]
    BB = 2048
    grid = (B // BB,)
    full = lambda shape: pl.BlockSpec(shape, lambda i: (0, 0))
    return pl.pallas_call(
        _mlp_body,
        grid=grid,
        in_specs=[
            pl.BlockSpec((BB, 2 * EMBED_DIM), lambda i: (i, 0)),
            pl.BlockSpec((BB, 2 * EMBED_DIM), lambda i: (i, 0)),
            pl.BlockSpec((BB, 1), lambda i: (i, 0)),
            pl.BlockSpec((BB, 1), lambda i: (i, 0)),
            full((EMBED_DIM, 128)),
            full((EMBED_DIM, 128)),
            full((1, 128)),
            full((128, 64)),
            full((1, 64)),
            full((1, 64)),
            full((1, 1)),
        ],
        out_specs=pl.BlockSpec((1, BB), lambda i: (0, i)),
        out_shape=jax.ShapeDtypeStruct((1, B), jnp.float32),
    )(gu, gp, qu, qp, W1u, W1p, b1, W2, b2, W3r, b3)


def kernel(user_ids, post_ids, user_table, post_table, W1, b1, W2, b2, W3, b3):
    B = user_ids.shape[0]
    V = user_table.shape[0]
    n_steps = (V + LANES_PER_STEP - 1) // LANES_PER_STEP
    Np = n_steps * GROUPS_PER_STEP * 128
    uids = user_ids.astype(jnp.int32)
    pids = post_ids.astype(jnp.int32)
    ur = ((uids // 512) * 128 + uids % 128).reshape(B // 128, 128)
    pr = ((pids // 512) * 128 + pids % 128).reshape(B // 128, 128)
    uq = ((uids // 128) % 4).reshape(B, 1)
    pq = ((pids // 128) % 4).reshape(B, 1)
    bf = jnp.bfloat16
    gather = _gather_fn(B, Np)
    utab2 = _repack(user_table.T, n_steps)
    gu = gather(ur, utab2)
    ptab2 = _repack(post_table.T, n_steps)
    gp = gather(pr, ptab2)
    out_t = _mlp(gu, gp, uq, pq,
                 W1[:EMBED_DIM].astype(bf), W1[EMBED_DIM:].astype(bf),
                 b1.reshape(1, -1), W2, b2.reshape(1, -1),
                 W3.reshape(1, -1), b3.reshape(1, 1))
    return out_t.T


# speedup vs baseline: 1.4983x; 1.4983x over previous
import functools

import jax
import jax.numpy as jnp
from jax import lax
from jax.experimental import pallas as pl
from jax.experimental.pallas import tpu as pltpu
from jax.experimental.pallas import tpu_sc as plsc

EMBED_DIM = 64
PAIRS_PER_STEP = 128
LANES_PER_STEP = PAIRS_PER_STEP * 256


def _repack_body(t_ref, o_ref):
    ii = lax.broadcasted_iota(jnp.int32, (128, 128), 0)
    jj = lax.broadcasted_iota(jnp.int32, (128, 128), 1)
    ident = (ii == jj).astype(jnp.float32)
    cn00 = (((0,), (0,)), ((), ()))
    for t in range(PAIRS_PER_STEP):
        pair = jnp.concatenate(
            [t_ref[:, (2 * t) * 128:(2 * t + 1) * 128],
             t_ref[:, (2 * t + 1) * 128:(2 * t + 2) * 128]], axis=0)
        o_ref[t * 128:(t + 1) * 128, :] = lax.dot_general(pair, ident, cn00)


def _repack(tabT, n_steps):
    return pl.pallas_call(
        _repack_body,
        grid=(n_steps,),
        in_specs=[pl.BlockSpec((EMBED_DIM, LANES_PER_STEP), lambda i: (0, i))],
        out_specs=pl.BlockSpec((PAIRS_PER_STEP * 128, 2 * EMBED_DIM),
                               lambda i: (i, 0)),
        out_shape=jax.ShapeDtypeStruct(
            (n_steps * PAIRS_PER_STEP * 128, 2 * EMBED_DIM), jnp.float32),
    )(tabT)


@functools.cache
def _gather_fn(B, Np):
    info = plsc.get_sparse_core_info()
    NC, NS = info.num_cores, info.num_subcores
    NW = NC * NS
    b_per_w = B // NW
    CH = 128
    k = b_per_w // CH
    mesh = plsc.VectorSubcoreMesh(core_axis_name="c", subcore_axis_name="s")

    @functools.partial(
        pl.kernel,
        mesh=mesh,
        compiler_params=pltpu.CompilerParams(
            use_tc_tiling_on_sc=True, needs_layout_passes=False),
        out_type=(
            jax.ShapeDtypeStruct((B, 2 * EMBED_DIM), jnp.float32),
            jax.ShapeDtypeStruct((B, 2 * EMBED_DIM), jnp.float32),
        ),
        scratch_types=[
            pltpu.VMEM((2 * k, CH), jnp.int32),
            pltpu.VMEM((CH, 2 * EMBED_DIM), jnp.float32),
            pltpu.VMEM((CH, 2 * EMBED_DIM), jnp.float32),
            pltpu.SemaphoreType.DMA,
            pltpu.SemaphoreType.DMA,
            pltpu.SemaphoreType.DMA,
            pltpu.SemaphoreType.DMA,
        ],
    )
    def gk(uidx_hbm, pidx_hbm, utab, ptab, uout, pout, idx_v, buf0, buf1,
           gsem0, gsem1, wsem0, wsem1):
        wid = lax.axis_index("s") * NC + lax.axis_index("c")
        base = wid * b_per_w
        pltpu.sync_copy(uidx_hbm.at[pl.ds(wid * k, k)], idx_v.at[pl.ds(0, k)])
        pltpu.sync_copy(pidx_hbm.at[pl.ds(wid * k, k)], idx_v.at[pl.ds(k, k)])
        jobs = [(utab, uout, j) for j in range(k)] + \
               [(ptab, pout, j) for j in range(k)]
        bufs = (buf0, buf1)
        gsems = (gsem0, gsem1)
        wsems = (wsem0, wsem1)
        gathers = [None, None]
        writes = [None, None]
        for i, (tab, _, j) in enumerate(jobs):
            p = i % 2
            if writes[p] is not None:
                writes[p].wait()
            gathers[p] = pltpu.async_copy(
                tab.at[idx_v.at[i]], bufs[p], gsems[p])
            if i >= 1:
                q = (i - 1) % 2
                gathers[q].wait()
                _, out_prev, j_prev = jobs[i - 1]
                writes[q] = pltpu.async_copy(
                    bufs[q], out_prev.at[pl.ds(base + j_prev * CH, CH)],
                    wsems[q])
        last = (2 * k - 1) % 2
        gathers[last].wait()
        _, out_last, j_last = jobs[-1]
        writes[last] = pltpu.async_copy(
            bufs[last], out_last.at[pl.ds(base + j_last * CH, CH)],
            wsems[last])
        writes[0].wait()
        writes[1].wait()

    return gk


def _mlp_body(gu, gp, hu, hp_, w1u, w1p, b1, w2, b2, w3r, b3, o):
    u = gu[:, :EMBED_DIM] * (1.0 - hu[...]) + gu[:, EMBED_DIM:] * hu[...]
    p = gp[:, :EMBED_DIM] * (1.0 - hp_[...]) + gp[:, EMBED_DIM:] * hp_[...]
    h = jnp.dot(u, w1u[...]) + jnp.dot(p, w1p[...]) + b1[...]
    h = jnp.maximum(h, 0.0)
    h = jnp.maximum(jnp.dot(h, w2[...]) + b2[...], 0.0)
    cn11 = (((1,), (1,)), ((), ()))
    o[...] = jax.nn.sigmoid(lax.dot_general(w3r[...], h, cn11) + b3[...])


def _mlp(gu, gp, hu, hp_, W1u, W1p, b1, W2, b2, W3r, b3):
    B = gu.shape[0]
    BB = 2048
    grid = (B // BB,)
    full = lambda shape: pl.BlockSpec(shape, lambda i: (0, 0))
    return pl.pallas_call(
        _mlp_body,
        grid=grid,
        in_specs=[
            pl.BlockSpec((BB, 2 * EMBED_DIM), lambda i: (i, 0)),
            pl.BlockSpec((BB, 2 * EMBED_DIM), lambda i: (i, 0)),
            pl.BlockSpec((BB, 1), lambda i: (i, 0)),
            pl.BlockSpec((BB, 1), lambda i: (i, 0)),
            full((EMBED_DIM, 128)),
            full((EMBED_DIM, 128)),
            full((1, 128)),
            full((128, 64)),
            full((1, 64)),
            full((1, 64)),
            full((1, 1)),
        ],
        out_specs=pl.BlockSpec((1, BB), lambda i: (0, i)),
        out_shape=jax.ShapeDtypeStruct((1, B), jnp.float32),
    )(gu, gp, hu, hp_, W1u, W1p, b1, W2, b2, W3r, b3)


def kernel(user_ids, post_ids, user_table, post_table, W1, b1, W2, b2, W3, b3):
    B = user_ids.shape[0]
    V = user_table.shape[0]
    n_steps = (V + LANES_PER_STEP - 1) // LANES_PER_STEP
    Np = n_steps * PAIRS_PER_STEP * 128
    uids = user_ids.astype(jnp.int32)
    pids = post_ids.astype(jnp.int32)
    ur = ((uids // 256) * 128 + uids % 128).reshape(B // 128, 128)
    pr = ((pids // 256) * 128 + pids % 128).reshape(B // 128, 128)
    uh = ((uids // 128) % 2).astype(jnp.float32).reshape(B, 1)
    ph = ((pids // 128) % 2).astype(jnp.float32).reshape(B, 1)
    utab2 = _repack(user_table.T, n_steps)
    ptab2 = _repack(post_table.T, n_steps)
    gu, gp = _gather_fn(B, Np)(ur, pr, utab2, ptab2)
    out_t = _mlp(gu, gp, uh, ph, W1[:EMBED_DIM], W1[EMBED_DIM:],
                 b1.reshape(1, -1), W2, b2.reshape(1, -1),
                 W3.reshape(1, -1), b3.reshape(1, 1))
    return out_t.T
